# baseline (device time: 45697 ns/iter reference)
import jax
import jax.numpy as jnp
from jax import lax
from jax.experimental import pallas as pl
from jax.experimental.pallas import tpu as pltpu

N_DEV = 4
EPS = 1e-5
ROWBLK = 128
BLK = 8
CHUNK = BLK * ROWBLK
LAG = 2
VMEM_LIMIT = 64 * 1024 * 1024


def kernel(x, gamma, beta):
    m, nl = x.shape
    n_global = nl * N_DEV
    nb = m // ROWBLK
    nchunks = m // CHUNK

    def in_copy(x_hbm, xv, in_sems, g):
        rows = pl.ds(g * CHUNK, CHUNK)
        return pltpu.make_async_copy(x_hbm.at[rows, :], xv.at[rows, :],
                                     in_sems.at[g])

    def stats_rdma(pstat, rbuf, send_sems, recv_sems, k, g, peer):
        return pltpu.make_async_remote_copy(
            src_ref=pstat.at[:, pl.ds(g * BLK, BLK), :],
            dst_ref=rbuf.at[k - 1, g],
            send_sem=send_sems.at[k - 1, g],
            recv_sem=recv_sems.at[k - 1, g],
            device_id=(peer,),
            device_id_type=pl.DeviceIdType.MESH,
        )

    def body(x_hbm, g_ref, b_ref, o_ref, xv, pstat, rbuf,
             in_sems, send_sems, recv_sems):
        my = lax.axis_index("i")
        g = pl.program_id(0)

        @pl.when(g == 0)
        def _prologue():
            for gg in range(nchunks):
                in_copy(x_hbm, xv, in_sems, gg).start()
            barrier = pltpu.get_barrier_semaphore()
            for k in range(1, N_DEV):
                peer = lax.rem(my + k, N_DEV)
                pl.semaphore_signal(
                    barrier, inc=1,
                    device_id=(peer,), device_id_type=pl.DeviceIdType.MESH,
                )
            pl.semaphore_wait(barrier, N_DEV - 1)

        @pl.when(g < nchunks)
        def _stats():
            in_copy(x_hbm, xv, in_sems, g).wait()
            xg = xv[pl.ds(g * CHUNK, CHUNK), :].reshape(BLK, ROWBLK, nl)
            sb = pl.ds(g * BLK, BLK)
            pstat[0, sb, :] = jnp.sum(xg, axis=2)
            pstat[1, sb, :] = jnp.sum(xg * xg, axis=2)
            for k in range(1, N_DEV):
                peer = lax.rem(my + k, N_DEV)
                stats_rdma(pstat, rbuf, send_sems, recv_sems, k, g, peer).start()

        @pl.when(g >= LAG)
        def _norm():
            gn = g - LAG
            for k in range(1, N_DEV):
                peer = lax.rem(my + k, N_DEV)
                stats_rdma(pstat, rbuf, send_sems, recv_sems,
                           k, gn, peer).wait_recv()
            sb = pl.ds(gn * BLK, BLK)
            s1 = pstat[0, sb, :] + rbuf[0, gn, 0] + rbuf[1, gn, 0] + rbuf[2, gn, 0]
            s2 = pstat[1, sb, :] + rbuf[0, gn, 1] + rbuf[1, gn, 1] + rbuf[2, gn, 1]
            mean = s1 / n_global
            var = s2 / n_global - mean * mean
            rstd = lax.rsqrt(var + EPS)
            mrs = mean * rstd

            xg = xv[pl.ds(gn * CHUNK, CHUNK), :].reshape(BLK, ROWBLK, nl)
            gg = g_ref[...].reshape(1, 1, nl)
            bb = b_ref[...].reshape(1, 1, nl)
            t = xg * rstd.reshape(BLK, ROWBLK, 1) - mrs.reshape(BLK, ROWBLK, 1)
            o = t * gg + bb
            o_ref[...] = o.reshape(CHUNK, nl).astype(o_ref.dtype)

        @pl.when(g == nchunks + LAG - 1)
        def _epilogue():
            for gg in range(nchunks):
                for k in range(1, N_DEV):
                    peer = lax.rem(my + k, N_DEV)
                    stats_rdma(pstat, rbuf, send_sems, recv_sems,
                               k, gg, peer).wait_send()

    return pl.pallas_call(
        body,
        grid=(nchunks + LAG,),
        in_specs=[
            pl.BlockSpec(memory_space=pl.ANY),
            pl.BlockSpec(memory_space=pltpu.VMEM),
            pl.BlockSpec(memory_space=pltpu.VMEM),
        ],
        out_specs=pl.BlockSpec(
            (CHUNK, nl), lambda g: (jnp.maximum(g - LAG, 0), 0)
        ),
        out_shape=jax.ShapeDtypeStruct((m, nl), jnp.bfloat16),
        scratch_shapes=[
            pltpu.VMEM((m, nl), jnp.float32),
            pltpu.VMEM((2, nb, ROWBLK), jnp.float32),
            pltpu.VMEM((N_DEV - 1, m // CHUNK, 2, BLK, ROWBLK), jnp.float32),
            pltpu.SemaphoreType.DMA((m // CHUNK,)),
            pltpu.SemaphoreType.DMA((N_DEV - 1, m // CHUNK)),
            pltpu.SemaphoreType.DMA((N_DEV - 1, m // CHUNK)),
        ],
        compiler_params=pltpu.CompilerParams(
            collective_id=0, vmem_limit_bytes=VMEM_LIMIT,
        ),
    )(x, gamma.reshape(1, nl), beta.reshape(1, nl))
